# baseline (speedup 1.0000x reference)
import functools

import jax
import jax.numpy as jnp
from jax import lax
from jax.experimental import pallas as pl
from jax.experimental.pallas import tpu as pltpu
from jax.experimental.pallas import tpu_sc as plsc

_N = 10000
_E = 320000
_D = 128
_NS = 16
_K = 128
_U = 160
_ST = 4
_CPS = _U // _ST
_TOT = _NS * _U
_ACC = 10112
_RPS = _ACC // _NS
_TRASH = _N + 7

_BLK = 1000


def _mlp_body(x_ref, w1_ref, b1_ref, w2_ref, b2_ref, o_ref):
    t = jnp.dot(x_ref[...], w1_ref[...], preferred_element_type=jnp.float32)
    t = jnp.maximum(t + b1_ref[...], 0.0)
    o_ref[...] = (
        jnp.dot(t, w2_ref[...], preferred_element_type=jnp.float32)
        + b2_ref[...]
    )


def _msg_mlp(h, w1, b1, w2, b2):
    full = lambda i: (0, 0)
    return pl.pallas_call(
        _mlp_body,
        grid=(_N // _BLK,),
        in_specs=[
            pl.BlockSpec((_BLK, _D), lambda i: (i, 0)),
            pl.BlockSpec((_D, _D), full),
            pl.BlockSpec((1, _D), full),
            pl.BlockSpec((_D, _D), full),
            pl.BlockSpec((1, _D), full),
        ],
        out_specs=pl.BlockSpec((_BLK, _D), lambda i: (i, 0)),
        out_shape=jax.ShapeDtypeStruct((_N, _D), jnp.float32),
    )(h, w1, b1.reshape(1, _D), w2, b2.reshape(1, _D))


def _upd_body(h_ref, p_ref, u1h_ref, u1a_ref, b1_ref, u2_ref, b2_ref, o_ref):
    t = jnp.dot(h_ref[...], u1h_ref[...], preferred_element_type=jnp.float32)
    t += jnp.dot(p_ref[...], u1a_ref[...], preferred_element_type=jnp.float32)
    t = jnp.maximum(t + b1_ref[...], 0.0)
    o_ref[...] = (
        jnp.dot(t, u2_ref[...], preferred_element_type=jnp.float32)
        + b2_ref[...]
    )


def _upd_mlp(h, aggr, u1h, u1a, b1, u2, b2):
    full = lambda i: (0, 0)
    return pl.pallas_call(
        _upd_body,
        grid=(_N // _BLK,),
        in_specs=[
            pl.BlockSpec((_BLK, _D), lambda i: (i, 0)),
            pl.BlockSpec((_BLK, _D), lambda i: (i, 0)),
            pl.BlockSpec((_D, _D), full),
            pl.BlockSpec((_D, _D), full),
            pl.BlockSpec((1, _D), full),
            pl.BlockSpec((_D, _D), full),
            pl.BlockSpec((1, _D), full),
        ],
        out_specs=pl.BlockSpec((_BLK, _D), lambda i: (i, 0)),
        out_shape=jax.ShapeDtypeStruct((_N, _D), jnp.float32),
    )(h, aggr, u1h, u1a, b1.reshape(1, _D), u2, b2.reshape(1, _D))


def _sc_edge_body(m_hbm, srcp_hbm, dstp_hbm, zeros_hbm, out_hbm,
                  acc, src_v, dst_v, rows0, rows1, sem0, sem1):
    s = lax.axis_index("s")
    base = s * _U

    pltpu.sync_copy(zeros_hbm.at[pl.ds(s * _RPS, _RPS)],
                    acc.at[pl.ds(s * _RPS, _RPS)])
    plsc.subcore_barrier()

    def gather(j, buf, sem):
        return pltpu.async_copy(m_hbm.at[src_v.at[j]], buf, sem)

    def wait_g(j, buf, sem):
        pltpu.make_async_copy(m_hbm.at[src_v.at[j]], buf, sem).wait()

    def scat(j, buf):
        pltpu.sync_copy(buf, acc.at[dst_v.at[j]], add=True)

    for st in range(_ST):
        pltpu.sync_copy(srcp_hbm.at[pl.ds(base + st * _CPS, _CPS)], src_v)
        pltpu.sync_copy(dstp_hbm.at[pl.ds(base + st * _CPS, _CPS)], dst_v)

        gather(0, rows0, sem0)

        @pl.loop(0, _CPS - 2, step=2)
        def _(g):
            gather(g + 1, rows1, sem1)
            wait_g(g, rows0, sem0)
            scat(g, rows0)
            gather(g + 2, rows0, sem0)
            wait_g(g + 1, rows1, sem1)
            scat(g + 1, rows1)

        gather(_CPS - 1, rows1, sem1)
        wait_g(_CPS - 2, rows0, sem0)
        scat(_CPS - 2, rows0)
        wait_g(_CPS - 1, rows1, sem1)
        scat(_CPS - 1, rows1)

    plsc.subcore_barrier()

    pltpu.sync_copy(acc.at[pl.ds(s * _RPS, _RPS)],
                    out_hbm.at[pl.ds(s * _RPS, _RPS)])


@functools.partial(
    pl.kernel,
    out_type=jax.ShapeDtypeStruct((_ACC, _D), jnp.float32),
    mesh=plsc.VectorSubcoreMesh(core_axis_name="c", subcore_axis_name="s",
                                num_cores=1),
    scratch_types=[
        pltpu.VMEM_SHARED((_ACC, _D), jnp.float32),
        pltpu.VMEM((_CPS, _K), jnp.int32),
        pltpu.VMEM((_CPS, _K), jnp.int32),
        pltpu.VMEM((_K, _D), jnp.float32),
        pltpu.VMEM((_K, _D), jnp.float32),
        pltpu.SemaphoreType.DMA,
        pltpu.SemaphoreType.DMA,
    ],
)
def _sc_edge(m_hbm, srcp_hbm, dstp_hbm, zeros_hbm, out_hbm,
             acc, src_v, dst_v, rows0, rows1, sem0, sem1):
    _sc_edge_body(m_hbm, srcp_hbm, dstp_hbm, zeros_hbm, out_hbm,
                  acc, src_v, dst_v, rows0, rows1, sem0, sem1)


def kernel(x, edge_index, batch, params):
    src = edge_index[0].astype(jnp.int32)
    dst = edge_index[1].astype(jnp.int32)
    pad = _TOT * _K - _E
    srcp = jnp.concatenate([src, jnp.zeros((pad,), jnp.int32)])
    srcp = srcp.reshape(_TOT, _K)
    dstp = jnp.concatenate([dst, jnp.full((pad,), _TRASH, jnp.int32)])
    dstp = dstp.reshape(_TOT, _K)
    zeros = jnp.zeros((_ACC, _D), jnp.float32)

    h = x
    for p in params:
        m = _msg_mlp(h, p['msg_W1'], p['msg_b1'], p['msg_W2'], p['msg_b2'])
        aggr = _sc_edge(m, srcp, dstp, zeros)
        h = _upd_mlp(h, aggr, p['upd_W1'][:_D], p['upd_W1'][_D:],
                     p['upd_b1'], p['upd_W2'], p['upd_b2'])
    return h


# speedup vs baseline: 1.1374x; 1.1374x over previous
import functools

import jax
import jax.numpy as jnp
from jax import lax
from jax.experimental import pallas as pl
from jax.experimental.pallas import tpu as pltpu
from jax.experimental.pallas import tpu_sc as plsc

_N = 10000
_E = 320000
_D = 128
_NC = 2
_NS = 16
_K = 128
_U0 = 128
_U1 = 32
_ST = 4
_CPS0 = _U0 // _ST
_CPS1 = _U1 // _ST
_TOT = _NS * (_U0 + _U1)
_ALLOC = _TOT + _CPS0
_ACC = 10112
_RPS = _ACC // _NS
_TRASH = _N + 7

_BLK = 1000


def _mlp_body(x_ref, w1_ref, b1_ref, w2_ref, b2_ref, o_ref):
    t = jnp.dot(x_ref[...], w1_ref[...], preferred_element_type=jnp.float32)
    t = jnp.maximum(t + b1_ref[...], 0.0)
    o_ref[...] = (
        jnp.dot(t, w2_ref[...], preferred_element_type=jnp.float32)
        + b2_ref[...]
    )


def _msg_mlp(h, w1, b1, w2, b2):
    full = lambda i: (0, 0)
    return pl.pallas_call(
        _mlp_body,
        grid=(_N // _BLK,),
        in_specs=[
            pl.BlockSpec((_BLK, _D), lambda i: (i, 0)),
            pl.BlockSpec((_D, _D), full),
            pl.BlockSpec((1, _D), full),
            pl.BlockSpec((_D, _D), full),
            pl.BlockSpec((1, _D), full),
        ],
        out_specs=pl.BlockSpec((_BLK, _D), lambda i: (i, 0)),
        out_shape=jax.ShapeDtypeStruct((_N, _D), jnp.float32),
    )(h, w1, b1.reshape(1, _D), w2, b2.reshape(1, _D))


def _upd_body(h_ref, p0_ref, p1_ref, u1h_ref, u1a_ref, b1_ref, u2_ref,
              b2_ref, o_ref):
    aggr = p0_ref[...] + p1_ref[...]
    t = jnp.dot(h_ref[...], u1h_ref[...], preferred_element_type=jnp.float32)
    t += jnp.dot(aggr, u1a_ref[...], preferred_element_type=jnp.float32)
    t = jnp.maximum(t + b1_ref[...], 0.0)
    o_ref[...] = (
        jnp.dot(t, u2_ref[...], preferred_element_type=jnp.float32)
        + b2_ref[...]
    )


def _upd_mlp(h, p0, p1, u1h, u1a, b1, u2, b2):
    full = lambda i: (0, 0)
    return pl.pallas_call(
        _upd_body,
        grid=(_N // _BLK,),
        in_specs=[
            pl.BlockSpec((_BLK, _D), lambda i: (i, 0)),
            pl.BlockSpec((_BLK, _D), lambda i: (i, 0)),
            pl.BlockSpec((_BLK, _D), lambda i: (i, 0)),
            pl.BlockSpec((_D, _D), full),
            pl.BlockSpec((_D, _D), full),
            pl.BlockSpec((1, _D), full),
            pl.BlockSpec((_D, _D), full),
            pl.BlockSpec((1, _D), full),
        ],
        out_specs=pl.BlockSpec((_BLK, _D), lambda i: (i, 0)),
        out_shape=jax.ShapeDtypeStruct((_N, _D), jnp.float32),
    )(h, p0, p1, u1h, u1a, b1.reshape(1, _D), u2, b2.reshape(1, _D))


def _sc_edge_body(m_hbm, srcp_hbm, dstp_hbm, zeros_hbm, out_hbm,
                  acc, src_v, dst_v, rows0, rows1, sem0, sem1):
    c = lax.axis_index("c")
    s = lax.axis_index("s")
    base = lax.select(c == 0, s * _U0, _NS * _U0 + s * _U1)
    cps = lax.select(c == 0, _CPS0, _CPS1)

    pltpu.sync_copy(zeros_hbm.at[pl.ds(s * _RPS, _RPS)],
                    acc.at[pl.ds(s * _RPS, _RPS)])
    plsc.subcore_barrier()

    def gather(j, buf, sem):
        return pltpu.async_copy(m_hbm.at[src_v.at[j]], buf, sem)

    def wait_g(j, buf, sem):
        pltpu.make_async_copy(m_hbm.at[src_v.at[j]], buf, sem).wait()

    def scat(j, buf):
        pltpu.sync_copy(buf, acc.at[dst_v.at[j]], add=True)

    for st in range(_ST):
        start = base + st * cps
        pltpu.sync_copy(srcp_hbm.at[pl.ds(start, _CPS0)], src_v)
        pltpu.sync_copy(dstp_hbm.at[pl.ds(start, _CPS0)], dst_v)

        gather(0, rows0, sem0)

        @pl.loop(0, cps - 2, step=2)
        def _(g):
            gather(g + 1, rows1, sem1)
            wait_g(g, rows0, sem0)
            scat(g, rows0)
            gather(g + 2, rows0, sem0)
            wait_g(g + 1, rows1, sem1)
            scat(g + 1, rows1)

        gather(cps - 1, rows1, sem1)
        wait_g(cps - 2, rows0, sem0)
        scat(cps - 2, rows0)
        wait_g(cps - 1, rows1, sem1)
        scat(cps - 1, rows1)

    plsc.subcore_barrier()

    pltpu.sync_copy(acc.at[pl.ds(s * _RPS, _RPS)],
                    out_hbm.at[c, pl.ds(s * _RPS, _RPS)])


@functools.partial(
    pl.kernel,
    out_type=jax.ShapeDtypeStruct((_NC, _ACC, _D), jnp.float32),
    mesh=plsc.VectorSubcoreMesh(core_axis_name="c", subcore_axis_name="s"),
    scratch_types=[
        pltpu.VMEM_SHARED((_ACC, _D), jnp.float32),
        pltpu.VMEM((_CPS0, _K), jnp.int32),
        pltpu.VMEM((_CPS0, _K), jnp.int32),
        pltpu.VMEM((_K, _D), jnp.float32),
        pltpu.VMEM((_K, _D), jnp.float32),
        pltpu.SemaphoreType.DMA,
        pltpu.SemaphoreType.DMA,
    ],
)
def _sc_edge(m_hbm, srcp_hbm, dstp_hbm, zeros_hbm, out_hbm,
             acc, src_v, dst_v, rows0, rows1, sem0, sem1):
    _sc_edge_body(m_hbm, srcp_hbm, dstp_hbm, zeros_hbm, out_hbm,
                  acc, src_v, dst_v, rows0, rows1, sem0, sem1)


def kernel(x, edge_index, batch, params):
    src = edge_index[0].astype(jnp.int32)
    dst = edge_index[1].astype(jnp.int32)
    pad = _ALLOC * _K - _E
    srcp = jnp.concatenate([src, jnp.zeros((pad,), jnp.int32)])
    srcp = srcp.reshape(_ALLOC, _K)
    dstp = jnp.concatenate([dst, jnp.full((pad,), _TRASH, jnp.int32)])
    dstp = dstp.reshape(_ALLOC, _K)
    zeros = jnp.zeros((_ACC, _D), jnp.float32)

    h = x
    for p in params:
        m = _msg_mlp(h, p['msg_W1'], p['msg_b1'], p['msg_W2'], p['msg_b2'])
        parts = _sc_edge(m, srcp, dstp, zeros)
        h = _upd_mlp(h, parts[0], parts[1],
                     p['upd_W1'][:_D], p['upd_W1'][_D:], p['upd_b1'],
                     p['upd_W2'], p['upd_b2'])
    return h


# speedup vs baseline: 1.4381x; 1.2644x over previous
import functools

import jax
import jax.numpy as jnp
from jax import lax
from jax.experimental import pallas as pl
from jax.experimental.pallas import tpu as pltpu
from jax.experimental.pallas import tpu_sc as plsc

_N = 10000
_E = 320000
_D = 128
_NC = 2
_NS = 16
_NW = _NC * _NS
_K = 128
_CH = 79
_EPW = _K * _CH
_ACC = 10240
_RPS = _ACC // _NS
_TRASH = _N + 7

_BLK = 1000


def _mlp_body(x_ref, w1_ref, b1_ref, w2_ref, b2_ref, o_ref):
    t = jnp.dot(x_ref[...], w1_ref[...], preferred_element_type=jnp.float32)
    t = jnp.maximum(t + b1_ref[...], 0.0)
    o_ref[...] = (
        jnp.dot(t, w2_ref[...], preferred_element_type=jnp.float32)
        + b2_ref[...]
    )


def _msg_mlp(h, w1, b1, w2, b2):
    full = lambda i: (0, 0)
    return pl.pallas_call(
        _mlp_body,
        grid=(_N // _BLK,),
        in_specs=[
            pl.BlockSpec((_BLK, _D), lambda i: (i, 0)),
            pl.BlockSpec((_D, _D), full),
            pl.BlockSpec((1, _D), full),
            pl.BlockSpec((_D, _D), full),
            pl.BlockSpec((1, _D), full),
        ],
        out_specs=pl.BlockSpec((_BLK, _D), lambda i: (i, 0)),
        out_shape=jax.ShapeDtypeStruct((_N, _D), jnp.float32),
    )(h, w1, b1.reshape(1, _D), w2, b2.reshape(1, _D))


def _upd_body(h_ref, p0_ref, p1_ref, u1h_ref, u1a_ref, b1_ref, u2_ref,
              b2_ref, o_ref):
    aggr = p0_ref[...] + p1_ref[...]
    t = jnp.dot(h_ref[...], u1h_ref[...], preferred_element_type=jnp.float32)
    t += jnp.dot(aggr, u1a_ref[...], preferred_element_type=jnp.float32)
    t = jnp.maximum(t + b1_ref[...], 0.0)
    o_ref[...] = (
        jnp.dot(t, u2_ref[...], preferred_element_type=jnp.float32)
        + b2_ref[...]
    )


def _upd_mlp(h, p0, p1, u1h, u1a, b1, u2, b2):
    full = lambda i: (0, 0)
    return pl.pallas_call(
        _upd_body,
        grid=(_N // _BLK,),
        in_specs=[
            pl.BlockSpec((_BLK, _D), lambda i: (i, 0)),
            pl.BlockSpec((_BLK, _D), lambda i: (i, 0)),
            pl.BlockSpec((_BLK, _D), lambda i: (i, 0)),
            pl.BlockSpec((_D, _D), full),
            pl.BlockSpec((_D, _D), full),
            pl.BlockSpec((1, _D), full),
            pl.BlockSpec((_D, _D), full),
            pl.BlockSpec((1, _D), full),
        ],
        out_specs=pl.BlockSpec((_BLK, _D), lambda i: (i, 0)),
        out_shape=jax.ShapeDtypeStruct((_N, _D), jnp.float32),
    )(h, p0, p1, u1h, u1a, b1.reshape(1, _D), u2, b2.reshape(1, _D))


def _sc_edge_body(m_hbm, srcp_hbm, dstp_hbm, zeros_hbm, out_hbm,
                  acc, src_v, dst_v, rows_v, sem):
    c = lax.axis_index("c")
    s = lax.axis_index("s")
    wid = s * _NC + c

    pltpu.sync_copy(zeros_hbm.at[pl.ds(s * _RPS, _RPS)],
                    acc.at[pl.ds(s * _RPS, _RPS)])

    pltpu.sync_copy(srcp_hbm.at[wid], src_v)
    pltpu.sync_copy(dstp_hbm.at[wid], dst_v)

    plsc.subcore_barrier()

    def chunk(j, carry):
        pltpu.async_copy(m_hbm.at[src_v.at[j]], rows_v, sem).wait()
        pltpu.sync_copy(rows_v, acc.at[dst_v.at[j]], add=True)
        return carry

    lax.fori_loop(0, _CH, chunk, 0)

    plsc.subcore_barrier()

    pltpu.sync_copy(acc.at[pl.ds(s * _RPS, _RPS)],
                    out_hbm.at[c, pl.ds(s * _RPS, _RPS)])


@functools.partial(
    pl.kernel,
    out_type=jax.ShapeDtypeStruct((_NC, _ACC, _D), jnp.float32),
    mesh=plsc.VectorSubcoreMesh(core_axis_name="c", subcore_axis_name="s"),
    scratch_types=[
        pltpu.VMEM_SHARED((_ACC, _D), jnp.float32),
        pltpu.VMEM((_CH, _K), jnp.int32),
        pltpu.VMEM((_CH, _K), jnp.int32),
        pltpu.VMEM((_K, _D), jnp.float32),
        pltpu.SemaphoreType.DMA,
    ],
)
def _sc_edge(m_hbm, srcp_hbm, dstp_hbm, zeros_hbm, out_hbm,
             acc, src_v, dst_v, rows_v, sem):
    _sc_edge_body(m_hbm, srcp_hbm, dstp_hbm, zeros_hbm, out_hbm,
                  acc, src_v, dst_v, rows_v, sem)


def kernel(x, edge_index, batch, params):
    src = edge_index[0].astype(jnp.int32)
    dst = edge_index[1].astype(jnp.int32)
    pad = _NW * _EPW - _E
    srcp = jnp.concatenate([src, jnp.zeros((pad,), jnp.int32)])
    srcp = srcp.reshape(_NW, _CH, _K)
    dstp = jnp.concatenate([dst, jnp.full((pad,), _TRASH, jnp.int32)])
    dstp = dstp.reshape(_NW, _CH, _K)
    zeros = jnp.zeros((_ACC, _D), jnp.float32)

    h = x
    for p in params:
        m = _msg_mlp(h, p['msg_W1'], p['msg_b1'], p['msg_W2'], p['msg_b2'])
        parts = _sc_edge(m, srcp, dstp, zeros)
        h = _upd_mlp(h, parts[0], parts[1],
                     p['upd_W1'][:_D], p['upd_W1'][_D:], p['upd_b1'],
                     p['upd_W2'], p['upd_b2'])
    return h


# speedup vs baseline: 1.5311x; 1.0646x over previous
import functools

import jax
import jax.numpy as jnp
from jax import lax
from jax.experimental import pallas as pl
from jax.experimental.pallas import tpu as pltpu
from jax.experimental.pallas import tpu_sc as plsc

_N = 10000
_E = 320000
_D = 128
_NC = 2
_NS = 16
_NW = _NC * _NS
_K = 128
_CH = 79
_EPW = _K * _CH
_ACC = 10240
_RPS = _ACC // _NS
_TRASH = _N + 7

_BLK = 1000


def _mlp_body(x_ref, w1_ref, b1_ref, w2_ref, b2_ref, o_ref):
    t = jnp.dot(x_ref[...], w1_ref[...], preferred_element_type=jnp.float32)
    t = jnp.maximum(t + b1_ref[...], 0.0)
    o_ref[...] = (
        jnp.dot(t, w2_ref[...], preferred_element_type=jnp.float32)
        + b2_ref[...]
    )


def _mlp2_body(x_ref, w1_ref, b1_ref, w2_ref, b2_ref, o_ref):
    t = jnp.dot(x_ref[...], w1_ref[...], preferred_element_type=jnp.float32)
    t = jnp.maximum(t + b1_ref[...], 0.0)
    o_ref[0] = (
        jnp.dot(t, w2_ref[...], preferred_element_type=jnp.float32)
        + b2_ref[...]
    )


def _msg_mlp(h, w1, b1, w2, b2):
    full = lambda c, i: (0, 0)
    return pl.pallas_call(
        _mlp2_body,
        grid=(_NC, _N // _BLK),
        in_specs=[
            pl.BlockSpec((_BLK, _D), lambda c, i: (i, 0)),
            pl.BlockSpec((_D, _D), full),
            pl.BlockSpec((1, _D), full),
            pl.BlockSpec((_D, _D), full),
            pl.BlockSpec((1, _D), full),
        ],
        out_specs=pl.BlockSpec((1, _BLK, _D), lambda c, i: (c, i, 0)),
        out_shape=jax.ShapeDtypeStruct((_NC, _N, _D), jnp.float32),
    )(h, w1, b1.reshape(1, _D), w2, b2.reshape(1, _D))


def _upd_body(h_ref, p_ref, u1h_ref, u1a_ref, b1_ref, u2_ref,
              b2_ref, o_ref):
    aggr = p_ref[0] + p_ref[1]
    t = jnp.dot(h_ref[...], u1h_ref[...], preferred_element_type=jnp.float32)
    t += jnp.dot(aggr, u1a_ref[...], preferred_element_type=jnp.float32)
    t = jnp.maximum(t + b1_ref[...], 0.0)
    o_ref[...] = (
        jnp.dot(t, u2_ref[...], preferred_element_type=jnp.float32)
        + b2_ref[...]
    )


def _upd_mlp(h, parts, u1h, u1a, b1, u2, b2):
    full = lambda i: (0, 0)
    return pl.pallas_call(
        _upd_body,
        grid=(_N // _BLK,),
        in_specs=[
            pl.BlockSpec((_BLK, _D), lambda i: (i, 0)),
            pl.BlockSpec((_NC, _BLK, _D), lambda i: (0, i, 0)),
            pl.BlockSpec((_D, _D), full),
            pl.BlockSpec((_D, _D), full),
            pl.BlockSpec((1, _D), full),
            pl.BlockSpec((_D, _D), full),
            pl.BlockSpec((1, _D), full),
        ],
        out_specs=pl.BlockSpec((_BLK, _D), lambda i: (i, 0)),
        out_shape=jax.ShapeDtypeStruct((_N, _D), jnp.float32),
    )(h, parts, u1h, u1a, b1.reshape(1, _D), u2, b2.reshape(1, _D))


def _sc_edge_body(m_hbm, srcp_hbm, dstp_hbm, zeros_hbm, out_hbm,
                  acc, src_v, dst_v, rows_v, sem):
    c = lax.axis_index("c")
    s = lax.axis_index("s")
    wid = s * _NC + c

    pltpu.sync_copy(zeros_hbm.at[pl.ds(s * _RPS, _RPS)],
                    acc.at[pl.ds(s * _RPS, _RPS)])

    pltpu.sync_copy(srcp_hbm.at[c, wid], src_v)
    pltpu.sync_copy(dstp_hbm.at[wid], dst_v)

    plsc.subcore_barrier()

    def chunk(j, carry):
        pltpu.async_copy(m_hbm.at[src_v.at[j]], rows_v, sem).wait()
        pltpu.sync_copy(rows_v, acc.at[dst_v.at[j]], add=True)
        return carry

    lax.fori_loop(0, _CH, chunk, 0)

    plsc.subcore_barrier()

    pltpu.sync_copy(acc.at[pl.ds(s * _RPS, _RPS)],
                    out_hbm.at[c, pl.ds(s * _RPS, _RPS)])


@functools.partial(
    pl.kernel,
    out_type=jax.ShapeDtypeStruct((_NC, _ACC, _D), jnp.float32),
    mesh=plsc.VectorSubcoreMesh(core_axis_name="c", subcore_axis_name="s"),
    scratch_types=[
        pltpu.VMEM_SHARED((_ACC, _D), jnp.float32),
        pltpu.VMEM((_CH, _K), jnp.int32),
        pltpu.VMEM((_CH, _K), jnp.int32),
        pltpu.VMEM((_K, _D), jnp.float32),
        pltpu.SemaphoreType.DMA,
    ],
)
def _sc_edge(m_hbm, srcp_hbm, dstp_hbm, zeros_hbm, out_hbm,
             acc, src_v, dst_v, rows_v, sem):
    _sc_edge_body(m_hbm, srcp_hbm, dstp_hbm, zeros_hbm, out_hbm,
                  acc, src_v, dst_v, rows_v, sem)


def kernel(x, edge_index, batch, params):
    src = edge_index[0].astype(jnp.int32)
    dst = edge_index[1].astype(jnp.int32)
    pad = _NW * _EPW - _E
    srcp = jnp.concatenate([src, jnp.zeros((pad,), jnp.int32)])
    srcp = srcp.reshape(_NW, _CH, _K)
    srcp = jnp.stack([srcp, srcp + _N])
    dstp = jnp.concatenate([dst, jnp.full((pad,), _TRASH, jnp.int32)])
    dstp = dstp.reshape(_NW, _CH, _K)
    zeros = jnp.zeros((_ACC, _D), jnp.float32)

    h = x
    for p in params:
        m = _msg_mlp(h, p['msg_W1'], p['msg_b1'], p['msg_W2'], p['msg_b2'])
        parts = _sc_edge(m.reshape(_NC * _N, _D), srcp, dstp, zeros)
        h = _upd_mlp(h, parts, p['upd_W1'][:_D], p['upd_W1'][_D:],
                     p['upd_b1'], p['upd_W2'], p['upd_b2'])
    return h
